# baseline (speedup 1.0000x reference)
import functools

import jax
import jax.numpy as jnp
from jax import lax
from jax.experimental import pallas as pl
from jax.experimental.pallas import tpu as pltpu
from jax.experimental.pallas import tpu_sc as plsc

L = 16

_GATHER_DN = lax.GatherDimensionNumbers(
    offset_dims=(), collapsed_slice_dims=(0,), start_index_map=(0,))


def _lanesum(v, lane):
    for sh in (8, 4, 2, 1):
        idx = lax.bitwise_xor(lane, sh)
        g = lax.gather(v, idx[:, None], dimension_numbers=_GATHER_DN,
                       slice_sizes=(1,),
                       mode=lax.GatherScatterMode.PROMISE_IN_BOUNDS)
        v = v + g
    return v


@functools.lru_cache(maxsize=None)
def _build_sc_kernel(B, K, D):
    info = plsc.get_sparse_core_info()
    NC, NS = info.num_cores, info.num_subcores
    NW = NC * NS
    BPW = B // NW
    C = 32
    NCHUNK = BPW // C
    CK = C * K
    IDXCH = 128
    NNEG = CK // IDXCH
    W = 2 * D
    DB = D // L

    mesh = plsc.VectorSubcoreMesh(core_axis_name="c", subcore_axis_name="s")

    @functools.partial(
        pl.kernel,
        mesh=mesh,
        compiler_params=pltpu.CompilerParams(use_tc_tiling_on_sc=False),
        out_type=[
            jax.ShapeDtypeStruct((B,), jnp.float32),
            jax.ShapeDtypeStruct((B * K,), jnp.float32),
        ],
        scratch_types=[
            pltpu.VMEM((C,), jnp.int32),
            pltpu.VMEM((C,), jnp.int32),
            pltpu.VMEM((CK,), jnp.int32),
            pltpu.VMEM((C + L,), jnp.int32),
            pltpu.VMEM((C + L,), jnp.int32),
            pltpu.VMEM((CK + L,), jnp.int32),
            pltpu.VMEM((C, W), jnp.float32),
            pltpu.VMEM((C, W), jnp.float32),
            pltpu.VMEM((CK, W), jnp.float32),
            pltpu.VMEM((C,), jnp.float32),
            pltpu.VMEM((CK,), jnp.float32),
            pltpu.SemaphoreType.DMA,
        ],
    )
    def k(cen_i_hbm, ctx_i_hbm, neg_i_hbm, cen_h_hbm, ctx_h_hbm, neg_h_hbm,
          cen_w, ctx_w,
          pos_hbm, negout_hbm,
          cen_i_v, ctx_i_v, neg_i_v, cen_h_v, ctx_h_v, neg_h_v,
          cen_r, ctx_r, neg_r, pos_v, neg_v, sem):
        wid = lax.axis_index("s") * NC + lax.axis_index("c")
        lane = lax.iota(jnp.int32, L)

        def chunk_body(ci, carry):
            base = wid * BPW + ci * C
            pltpu.sync_copy(cen_i_hbm.at[pl.ds(base, C)], cen_i_v)
            pltpu.sync_copy(ctx_i_hbm.at[pl.ds(base, C)], ctx_i_v)
            pltpu.sync_copy(neg_i_hbm.at[pl.ds(base * K, CK)], neg_i_v)
            pltpu.sync_copy(cen_h_hbm.at[pl.ds(base, C)],
                            cen_h_v.at[pl.ds(0, C)])
            pltpu.sync_copy(ctx_h_hbm.at[pl.ds(base, C)],
                            ctx_h_v.at[pl.ds(0, C)])
            pltpu.sync_copy(neg_h_hbm.at[pl.ds(base * K, CK)],
                            neg_h_v.at[pl.ds(0, CK)])
            cps = [
                pltpu.async_copy(cen_w.at[cen_i_v], cen_r, sem),
                pltpu.async_copy(ctx_w.at[ctx_i_v], ctx_r, sem),
            ]
            for j in range(NNEG):
                cps.append(pltpu.async_copy(
                    ctx_w.at[neg_i_v.at[pl.ds(j * IDXCH, IDXCH)]],
                    neg_r.at[pl.ds(j * IDXCH, IDXCH)], sem))
            for cp in cps:
                cp.wait()

            GR = 4
            NV = GR * K // L

            def g_body(bg, pos_vec):
                nvs = [jnp.zeros((L,), jnp.float32) for _ in range(NV)]
                for j in range(GR):
                    b = bg * GR + j
                    ch = cen_h_v[pl.ds(b, L)][0]
                    th = ctx_h_v[pl.ds(b, L)][0]
                    c = [cen_r[b, pl.ds(ch + q * L, L)] for q in range(DB)]
                    t = [ctx_r[b, pl.ds(th + q * L, L)] for q in range(DB)]
                    acc = c[0] * t[0]
                    for q in range(1, DB):
                        acc = acc + c[q] * t[q]
                    ps = _lanesum(acc, lane)
                    pos_vec = jnp.where(lane == b % L, ps, pos_vec)
                    for kk in range(K):
                        r = b * K + kk
                        nh = neg_h_v[pl.ds(r, L)][0]
                        n = [neg_r[r, pl.ds(nh + q * L, L)]
                             for q in range(DB)]
                        na = c[0] * n[0]
                        for q in range(1, DB):
                            na = na + c[q] * n[q]
                        s = _lanesum(na, lane)
                        p = j * K + kk
                        nvs[p // L] = jnp.where(lane == p % L, s, nvs[p // L])
                for v in range(NV):
                    neg_v[pl.ds(bg * (GR * K) + v * L, L)] = nvs[v]

                @pl.when(bg % GR == GR - 1)
                def _():
                    pos_v[pl.ds(bg * GR - (L - GR), L)] = pos_vec

                return pos_vec

            lax.fori_loop(0, C // GR, g_body, jnp.zeros((L,), jnp.float32))
            pltpu.sync_copy(pos_v, pos_hbm.at[pl.ds(base, C)])
            pltpu.sync_copy(neg_v, negout_hbm.at[pl.ds(base * K, CK)])
            return carry

        lax.fori_loop(0, NCHUNK, chunk_body, jnp.int32(0))

    return k


def kernel(central_idxs, context_idxs, negative_samples_idxs,
           central_weight, context_weight):
    B, K = negative_samples_idxs.shape
    V, D = central_weight.shape
    cen = central_idxs.astype(jnp.int32)
    ctx = context_idxs.astype(jnp.int32)
    neg = negative_samples_idxs.astype(jnp.int32).reshape(B * K)
    cen_w2 = central_weight.reshape(V // 2, 2 * D)
    ctx_w2 = context_weight.reshape(V // 2, 2 * D)
    f = _build_sc_kernel(B, K, D)
    pos, negs = f(cen >> 1, ctx >> 1, neg >> 1,
                  (cen & 1) * D, (ctx & 1) * D, (neg & 1) * D,
                  cen_w2, ctx_w2)
    return pos.reshape(B, 1), negs.reshape(B, K)


# speedup vs baseline: 1.3447x; 1.3447x over previous
import functools

import jax
import jax.numpy as jnp
from jax import lax
from jax.experimental import pallas as pl
from jax.experimental.pallas import tpu as pltpu
from jax.experimental.pallas import tpu_sc as plsc

L = 16

_GATHER_DN = lax.GatherDimensionNumbers(
    offset_dims=(), collapsed_slice_dims=(0,), start_index_map=(0,))


def _lanesum(v, lane):
    for sh in (8, 4, 2, 1):
        idx = lax.bitwise_xor(lane, sh)
        g = lax.gather(v, idx[:, None], dimension_numbers=_GATHER_DN,
                       slice_sizes=(1,),
                       mode=lax.GatherScatterMode.PROMISE_IN_BOUNDS)
        v = v + g
    return v


@functools.lru_cache(maxsize=None)
def _build_sc_kernel(B, K, D):
    info = plsc.get_sparse_core_info()
    NC, NS = info.num_cores, info.num_subcores
    NW = NC * NS
    BPW = B // NW
    C = 32
    NCHUNK = BPW // C
    CK = C * K
    DB = D // L

    mesh = plsc.VectorSubcoreMesh(core_axis_name="c", subcore_axis_name="s")

    @functools.partial(
        pl.kernel,
        mesh=mesh,
        compiler_params=pltpu.CompilerParams(use_tc_tiling_on_sc=True),
        out_type=[
            jax.ShapeDtypeStruct((B,), jnp.float32),
            jax.ShapeDtypeStruct((B * K,), jnp.float32),
        ],
        scratch_types=[
            pltpu.VMEM((C + L,), jnp.int32),
            pltpu.VMEM((C + L,), jnp.int32),
            pltpu.VMEM((CK + L,), jnp.int32),
            pltpu.VMEM((C, D), jnp.float32),
            pltpu.VMEM((C, D), jnp.float32),
            pltpu.VMEM((CK, D), jnp.float32),
            pltpu.VMEM((C,), jnp.float32),
            pltpu.VMEM((CK,), jnp.float32),
            pltpu.SemaphoreType.DMA,
        ],
    )
    def k(cen_i_hbm, ctx_i_hbm, neg_i_hbm, cen_w, ctx_w,
          pos_hbm, negout_hbm,
          cen_i_v, ctx_i_v, neg_i_v, cen_r, ctx_r, neg_r, pos_v, neg_v, sem):
        wid = lax.axis_index("s") * NC + lax.axis_index("c")
        lane = lax.iota(jnp.int32, L)

        def fetch_rows(idx_v, n, table, dst):
            def body(r, carry):
                i = idx_v[pl.ds(r, L)][0]
                pltpu.async_copy(table.at[pl.ds(i, 1), :],
                                 dst.at[pl.ds(r, 1), :], sem)
                return carry
            lax.fori_loop(0, n, body, jnp.int32(0))

        def drain(dst, table):
            pltpu.make_async_copy(
                table.at[pl.ds(0, dst.shape[0]), :], dst, sem).wait()

        def chunk_body(ci, carry):
            base = wid * BPW + ci * C
            pltpu.sync_copy(cen_i_hbm.at[pl.ds(base, C)],
                            cen_i_v.at[pl.ds(0, C)])
            pltpu.sync_copy(ctx_i_hbm.at[pl.ds(base, C)],
                            ctx_i_v.at[pl.ds(0, C)])
            pltpu.sync_copy(neg_i_hbm.at[pl.ds(base * K, CK)],
                            neg_i_v.at[pl.ds(0, CK)])
            fetch_rows(cen_i_v, C, cen_w, cen_r)
            fetch_rows(ctx_i_v, C, ctx_w, ctx_r)
            fetch_rows(neg_i_v, CK, ctx_w, neg_r)
            drain(cen_r, cen_w)
            drain(ctx_r, ctx_w)
            drain(neg_r, ctx_w)

            GR = 4
            NV = GR * K // L

            def g_body(bg, pos_vec):
                nvs = [jnp.zeros((L,), jnp.float32) for _ in range(NV)]
                for j in range(GR):
                    b = bg * GR + j
                    c = [cen_r[b, pl.ds(q * L, L)] for q in range(DB)]
                    t = [ctx_r[b, pl.ds(q * L, L)] for q in range(DB)]
                    acc = c[0] * t[0]
                    for q in range(1, DB):
                        acc = acc + c[q] * t[q]
                    ps = _lanesum(acc, lane)
                    pos_vec = jnp.where(lane == b % L, ps, pos_vec)
                    for kk in range(K):
                        r = b * K + kk
                        n = [neg_r[r, pl.ds(q * L, L)] for q in range(DB)]
                        na = c[0] * n[0]
                        for q in range(1, DB):
                            na = na + c[q] * n[q]
                        s = _lanesum(na, lane)
                        p = j * K + kk
                        nvs[p // L] = jnp.where(lane == p % L, s, nvs[p // L])
                for v in range(NV):
                    neg_v[pl.ds(bg * (GR * K) + v * L, L)] = nvs[v]

                @pl.when(bg % GR == GR - 1)
                def _():
                    pos_v[pl.ds(bg * GR - (L - GR), L)] = pos_vec

                return pos_vec

            lax.fori_loop(0, C // GR, g_body, jnp.zeros((L,), jnp.float32))
            pltpu.sync_copy(pos_v, pos_hbm.at[pl.ds(base, C)])
            pltpu.sync_copy(neg_v, negout_hbm.at[pl.ds(base * K, CK)])
            return carry

        lax.fori_loop(0, NCHUNK, chunk_body, jnp.int32(0))

    return k


def kernel(central_idxs, context_idxs, negative_samples_idxs,
           central_weight, context_weight):
    B, K = negative_samples_idxs.shape
    _, D = central_weight.shape
    cen = central_idxs.astype(jnp.int32)
    ctx = context_idxs.astype(jnp.int32)
    neg = negative_samples_idxs.astype(jnp.int32).reshape(B * K)
    f = _build_sc_kernel(B, K, D)
    pos, negs = f(cen, ctx, neg, central_weight, context_weight)
    return pos.reshape(B, 1), negs.reshape(B, K)


# speedup vs baseline: 1.8387x; 1.3674x over previous
import functools

import jax
import jax.numpy as jnp
from jax import lax
from jax.experimental import pallas as pl
from jax.experimental.pallas import tpu as pltpu
from jax.experimental.pallas import tpu_sc as plsc

L = 16

_GATHER_DN = lax.GatherDimensionNumbers(
    offset_dims=(), collapsed_slice_dims=(0,), start_index_map=(0,))


def _dyng(v, idx):
    return lax.gather(v, idx[:, None], dimension_numbers=_GATHER_DN,
                      slice_sizes=(1,),
                      mode=lax.GatherScatterMode.PROMISE_IN_BOUNDS)


def _lanesum(v, lane):
    for sh in (8, 4, 2, 1):
        v = v + _dyng(v, lax.bitwise_xor(lane, sh))
    return v


@functools.lru_cache(maxsize=None)
def _build_central_kernel(B, D):
    info = plsc.get_sparse_core_info()
    NC, NS = info.num_cores, info.num_subcores
    NW = NC * NS
    CPW = B // NW
    W = 128
    GB = 4
    NG = CPW // GB
    DB = D // L

    mesh = plsc.VectorSubcoreMesh(core_axis_name="c", subcore_axis_name="s")

    @functools.partial(
        pl.kernel,
        mesh=mesh,
        compiler_params=pltpu.CompilerParams(use_tc_tiling_on_sc=True),
        out_type=[jax.ShapeDtypeStruct((B * D,), jnp.float32)],
        scratch_types=[
            pltpu.VMEM((CPW + L,), jnp.int32),
            pltpu.VMEM((D, GB * W), jnp.float32),
            pltpu.VMEM((D, GB * W), jnp.float32),
            pltpu.VMEM((CPW * D,), jnp.float32),
            pltpu.SemaphoreType.DMA,
        ],
    )
    def k(idx_hbm, wt_hbm, out_hbm, idx_v, buf_a, buf_b, emb_v, sem):
        wid = lax.axis_index("s") * NC + lax.axis_index("c")
        lane = lax.iota(jnp.int32, L)
        base = wid * CPW
        pltpu.sync_copy(idx_hbm.at[pl.ds(base, CPW)], idx_v.at[pl.ds(0, CPW)])

        def fetch_group(g, buf):
            for j in range(GB):
                i = idx_v[pl.ds(g * GB + j, L)][0]
                blk = pl.multiple_of((i >> 7) << 7, 128)
                pltpu.async_copy(wt_hbm.at[:, pl.ds(blk, W)],
                                 buf.at[:, pl.ds(j * W, W)], sem)

        def drain_group(buf):
            pltpu.make_async_copy(wt_hbm.at[:, pl.ds(0, GB * W)], buf,
                                  sem).wait()

        def extract_group(g, buf):
            for j in range(GB):
                r = g * GB + j
                i = idx_v[pl.ds(r, L)][0]
                o = lax.broadcast(i & (L - 1), (L,))
                sub = pl.multiple_of(j * W + (((i >> 4) & 7) << 4), L)
                for q in range(DB):
                    acc = jnp.zeros((L,), jnp.float32)
                    for dd in range(L):
                        v = buf[q * L + dd, pl.ds(sub, L)]
                        acc = jnp.where(lane == dd, _dyng(v, o), acc)
                    emb_v[pl.ds(r * D + q * L, L)] = acc

        bufs = (buf_a, buf_b)
        fetch_group(0, bufs[0])

        def g_body(g, carry):
            @pl.when(g % 2 == 0)
            def _():
                drain_group(bufs[0])

                @pl.when(g + 1 < NG)
                def _():
                    fetch_group(g + 1, bufs[1])
                extract_group(g, bufs[0])

            @pl.when(g % 2 == 1)
            def _():
                drain_group(bufs[1])

                @pl.when(g + 1 < NG)
                def _():
                    fetch_group(g + 1, bufs[0])
                extract_group(g, bufs[1])
            return carry

        lax.fori_loop(0, NG, g_body, jnp.int32(0))
        pltpu.sync_copy(emb_v, out_hbm.at[pl.ds(base * D, CPW * D)])

    return k


@functools.lru_cache(maxsize=None)
def _build_main_kernel(B, K, D):
    info = plsc.get_sparse_core_info()
    NC, NS = info.num_cores, info.num_subcores
    NW = NC * NS
    BPW = B // NW
    C = 32
    NCHUNK = BPW // C
    CK = C * K
    DB = D // L

    mesh = plsc.VectorSubcoreMesh(core_axis_name="c", subcore_axis_name="s")

    @functools.partial(
        pl.kernel,
        mesh=mesh,
        compiler_params=pltpu.CompilerParams(use_tc_tiling_on_sc=True),
        out_type=[
            jax.ShapeDtypeStruct((B,), jnp.float32),
            jax.ShapeDtypeStruct((B * K,), jnp.float32),
        ],
        scratch_types=[
            pltpu.VMEM((C + L,), jnp.int32),
            pltpu.VMEM((CK + L,), jnp.int32),
            pltpu.VMEM((C * D,), jnp.float32),
            pltpu.VMEM((C, D), jnp.float32),
            pltpu.VMEM((CK, D), jnp.float32),
            pltpu.VMEM((C,), jnp.float32),
            pltpu.VMEM((CK,), jnp.float32),
            pltpu.SemaphoreType.DMA,
        ],
    )
    def k(ctx_i_hbm, neg_i_hbm, cemb_hbm, ctx_w,
          pos_hbm, negout_hbm,
          ctx_i_v, neg_i_v, cen_r, ctx_r, neg_r, pos_v, neg_v, sem):
        wid = lax.axis_index("s") * NC + lax.axis_index("c")
        lane = lax.iota(jnp.int32, L)

        def fetch_rows(idx_v, n, table, dst):
            def body(r, carry):
                i = idx_v[pl.ds(r, L)][0]
                pltpu.async_copy(table.at[pl.ds(i, 1), :],
                                 dst.at[pl.ds(r, 1), :], sem)
                return carry
            lax.fori_loop(0, n, body, jnp.int32(0))

        def drain(dst, table):
            pltpu.make_async_copy(
                table.at[pl.ds(0, dst.shape[0]), :], dst, sem).wait()

        def chunk_body(ci, carry):
            base = wid * BPW + ci * C
            pltpu.sync_copy(ctx_i_hbm.at[pl.ds(base, C)],
                            ctx_i_v.at[pl.ds(0, C)])
            pltpu.sync_copy(neg_i_hbm.at[pl.ds(base * K, CK)],
                            neg_i_v.at[pl.ds(0, CK)])
            pltpu.sync_copy(cemb_hbm.at[pl.ds(base * D, C * D)], cen_r)
            fetch_rows(ctx_i_v, C, ctx_w, ctx_r)
            fetch_rows(neg_i_v, CK, ctx_w, neg_r)
            drain(ctx_r, ctx_w)
            drain(neg_r, ctx_w)

            GR = 4
            NV = GR * K // L

            def g_body(bg, pos_vec):
                nvs = [jnp.zeros((L,), jnp.float32) for _ in range(NV)]
                for j in range(GR):
                    b = bg * GR + j
                    c = [cen_r[pl.ds(b * D + q * L, L)] for q in range(DB)]
                    t = [ctx_r[b, pl.ds(q * L, L)] for q in range(DB)]
                    acc = c[0] * t[0]
                    for q in range(1, DB):
                        acc = acc + c[q] * t[q]
                    ps = _lanesum(acc, lane)
                    pos_vec = jnp.where(lane == b % L, ps, pos_vec)
                    for kk in range(K):
                        r = b * K + kk
                        n = [neg_r[r, pl.ds(q * L, L)] for q in range(DB)]
                        na = c[0] * n[0]
                        for q in range(1, DB):
                            na = na + c[q] * n[q]
                        s = _lanesum(na, lane)
                        p = j * K + kk
                        nvs[p // L] = jnp.where(lane == p % L, s, nvs[p // L])
                for v in range(NV):
                    neg_v[pl.ds(bg * (GR * K) + v * L, L)] = nvs[v]

                @pl.when(bg % GR == GR - 1)
                def _():
                    pos_v[pl.ds(bg * GR - (L - GR), L)] = pos_vec

                return pos_vec

            lax.fori_loop(0, C // GR, g_body, jnp.zeros((L,), jnp.float32))
            pltpu.sync_copy(pos_v, pos_hbm.at[pl.ds(base, C)])
            pltpu.sync_copy(neg_v, negout_hbm.at[pl.ds(base * K, CK)])
            return carry

        lax.fori_loop(0, NCHUNK, chunk_body, jnp.int32(0))

    return k


def kernel(central_idxs, context_idxs, negative_samples_idxs,
           central_weight, context_weight):
    B, K = negative_samples_idxs.shape
    _, D = central_weight.shape
    cen = central_idxs.astype(jnp.int32)
    ctx = context_idxs.astype(jnp.int32)
    neg = negative_samples_idxs.astype(jnp.int32).reshape(B * K)
    cen_wt = jnp.transpose(central_weight)
    fc = _build_central_kernel(B, D)
    (cemb,) = fc(cen, cen_wt)
    fm = _build_main_kernel(B, K, D)
    pos, negs = fm(ctx, neg, cemb, context_weight)
    return pos.reshape(B, 1), negs.reshape(B, K)


# speedup vs baseline: 1.9149x; 1.0415x over previous
import functools

import jax
import jax.numpy as jnp
from jax import lax
from jax.experimental import pallas as pl
from jax.experimental.pallas import tpu as pltpu
from jax.experimental.pallas import tpu_sc as plsc

L = 16

_GATHER_DN = lax.GatherDimensionNumbers(
    offset_dims=(), collapsed_slice_dims=(0,), start_index_map=(0,))


def _dyng(v, idx):
    return lax.gather(v, idx[:, None], dimension_numbers=_GATHER_DN,
                      slice_sizes=(1,),
                      mode=lax.GatherScatterMode.PROMISE_IN_BOUNDS)


def _lanesum(v, lane):
    for sh in (8, 4, 2, 1):
        v = v + _dyng(v, lax.bitwise_xor(lane, sh))
    return v


@functools.lru_cache(maxsize=None)
def _build_central_kernel(B, D):
    info = plsc.get_sparse_core_info()
    NC, NS = info.num_cores, info.num_subcores
    NW = NC * NS
    CPW = B // NW
    W = 128
    GB = 4
    NG = CPW // GB
    DB = D // L

    mesh = plsc.VectorSubcoreMesh(core_axis_name="c", subcore_axis_name="s")

    @functools.partial(
        pl.kernel,
        mesh=mesh,
        compiler_params=pltpu.CompilerParams(use_tc_tiling_on_sc=True),
        out_type=[jax.ShapeDtypeStruct((B * D,), jnp.float32)],
        scratch_types=[
            pltpu.VMEM((CPW + L,), jnp.int32),
            pltpu.VMEM((D, GB * W), jnp.float32),
            pltpu.VMEM((D, GB * W), jnp.float32),
            pltpu.VMEM((CPW * D,), jnp.float32),
            pltpu.SemaphoreType.DMA,
        ],
    )
    def k(idx_hbm, wt_hbm, out_hbm, idx_v, buf_a, buf_b, emb_v, sem):
        wid = lax.axis_index("s") * NC + lax.axis_index("c")
        lane = lax.iota(jnp.int32, L)
        base = wid * CPW
        pltpu.sync_copy(idx_hbm.at[pl.ds(base, CPW)], idx_v.at[pl.ds(0, CPW)])

        def fetch_group(g, buf):
            for j in range(GB):
                i = idx_v[pl.ds(g * GB + j, L)][0]
                blk = pl.multiple_of((i >> 7) << 7, 128)
                pltpu.async_copy(wt_hbm.at[:, pl.ds(blk, W)],
                                 buf.at[:, pl.ds(j * W, W)], sem)

        def drain_group(buf):
            pltpu.make_async_copy(wt_hbm.at[:, pl.ds(0, GB * W)], buf,
                                  sem).wait()

        def extract_group(g, buf):
            for j in range(GB):
                r = g * GB + j
                i = idx_v[pl.ds(r, L)][0]
                o = lax.broadcast(i & (L - 1), (L,))
                sub = pl.multiple_of(j * W + (((i >> 4) & 7) << 4), L)
                for q in range(DB):
                    acc = jnp.zeros((L,), jnp.float32)
                    for dd in range(L):
                        v = buf[q * L + dd, pl.ds(sub, L)]
                        acc = jnp.where(lane == dd, _dyng(v, o), acc)
                    emb_v[pl.ds(r * D + q * L, L)] = acc

        bufs = (buf_a, buf_b)
        fetch_group(0, bufs[0])

        def g_body(g, carry):
            @pl.when(g % 2 == 0)
            def _():
                drain_group(bufs[0])

                @pl.when(g + 1 < NG)
                def _():
                    fetch_group(g + 1, bufs[1])
                extract_group(g, bufs[0])

            @pl.when(g % 2 == 1)
            def _():
                drain_group(bufs[1])

                @pl.when(g + 1 < NG)
                def _():
                    fetch_group(g + 1, bufs[0])
                extract_group(g, bufs[1])
            return carry

        lax.fori_loop(0, NG, g_body, jnp.int32(0))
        pltpu.sync_copy(emb_v, out_hbm.at[pl.ds(base * D, CPW * D)])

    return k


@functools.lru_cache(maxsize=None)
def _build_main_kernel(B, K, D):
    info = plsc.get_sparse_core_info()
    NC, NS = info.num_cores, info.num_subcores
    NW = NC * NS
    BPW = B // NW
    C = 16
    NCHUNK = BPW // C
    CK = C * K
    PK = BPW * K
    DB = D // L

    mesh = plsc.VectorSubcoreMesh(core_axis_name="c", subcore_axis_name="s")

    @functools.partial(
        pl.kernel,
        mesh=mesh,
        compiler_params=pltpu.CompilerParams(use_tc_tiling_on_sc=True),
        out_type=[
            jax.ShapeDtypeStruct((B,), jnp.float32),
            jax.ShapeDtypeStruct((B * K,), jnp.float32),
        ],
        scratch_types=[
            pltpu.VMEM((BPW + L,), jnp.int32),
            pltpu.VMEM((PK + L,), jnp.int32),
            pltpu.VMEM((C * D,), jnp.float32),
            pltpu.VMEM((C * D,), jnp.float32),
            pltpu.VMEM((C, D), jnp.float32),
            pltpu.VMEM((C, D), jnp.float32),
            pltpu.VMEM((CK, D), jnp.float32),
            pltpu.VMEM((CK, D), jnp.float32),
            pltpu.VMEM((BPW,), jnp.float32),
            pltpu.VMEM((PK,), jnp.float32),
            pltpu.SemaphoreType.DMA,
            pltpu.SemaphoreType.DMA,
        ],
    )
    def k(ctx_i_hbm, neg_i_hbm, cemb_hbm, ctx_w,
          pos_hbm, negout_hbm,
          ctx_i_v, neg_i_v, cen_a, cen_b, ctx_a, ctx_b, neg_a, neg_b,
          pos_v, neg_v, sem_a, sem_b):
        wid = lax.axis_index("s") * NC + lax.axis_index("c")
        lane = lax.iota(jnp.int32, L)
        base = wid * BPW
        pltpu.sync_copy(ctx_i_hbm.at[pl.ds(base, BPW)],
                        ctx_i_v.at[pl.ds(0, BPW)])
        pltpu.sync_copy(neg_i_hbm.at[pl.ds(base * K, PK)],
                        neg_i_v.at[pl.ds(0, PK)])

        cen_bufs = (cen_a, cen_b)
        ctx_bufs = (ctx_a, ctx_b)
        neg_bufs = (neg_a, neg_b)
        sems = (sem_a, sem_b)

        def fetch_rows(idx_v, off, n, dst, sem):
            def body(r4, carry):
                for u in range(4):
                    r = r4 * 4 + u
                    i = idx_v[pl.ds(off + r, L)][0]
                    pltpu.async_copy(ctx_w.at[pl.ds(i, 1), :],
                                     dst.at[pl.ds(r, 1), :], sem)
                return carry
            lax.fori_loop(0, n // 4, body, jnp.int32(0))

        def issue(ci, par):
            sem = sems[par]
            pltpu.async_copy(
                cemb_hbm.at[pl.ds((base + ci * C) * D, C * D)],
                cen_bufs[par], sem)
            fetch_rows(ctx_i_v, ci * C, C, ctx_bufs[par], sem)
            fetch_rows(neg_i_v, ci * CK, CK, neg_bufs[par], sem)

        def drain(par):
            sem = sems[par]
            pltpu.make_async_copy(cemb_hbm.at[pl.ds(0, C * D)],
                                  cen_bufs[par], sem).wait()
            pltpu.make_async_copy(ctx_w.at[pl.ds(0, C), :],
                                  ctx_bufs[par], sem).wait()
            pltpu.make_async_copy(ctx_w.at[pl.ds(0, CK), :],
                                  neg_bufs[par], sem).wait()

        GR = 4
        NV = GR * K // L

        def compute(ci, par):
            cen_r, ctx_r, neg_r = cen_bufs[par], ctx_bufs[par], neg_bufs[par]

            def g_body(bg, pos_vec):
                nvs = [jnp.zeros((L,), jnp.float32) for _ in range(NV)]
                for j in range(GR):
                    b = bg * GR + j
                    c = [cen_r[pl.ds(b * D + q * L, L)] for q in range(DB)]
                    t = [ctx_r[b, pl.ds(q * L, L)] for q in range(DB)]
                    acc = c[0] * t[0]
                    for q in range(1, DB):
                        acc = acc + c[q] * t[q]
                    ps = _lanesum(acc, lane)
                    pos_vec = jnp.where(lane == b % L, ps, pos_vec)
                    for kk in range(K):
                        r = b * K + kk
                        n = [neg_r[r, pl.ds(q * L, L)] for q in range(DB)]
                        na = c[0] * n[0]
                        for q in range(1, DB):
                            na = na + c[q] * n[q]
                        s = _lanesum(na, lane)
                        p = j * K + kk
                        nvs[p // L] = jnp.where(lane == p % L, s,
                                                nvs[p // L])
                for v in range(NV):
                    neg_v[pl.ds(ci * CK + bg * (GR * K) + v * L, L)] = nvs[v]
                return pos_vec

            pos_vec = lax.fori_loop(0, C // GR, g_body,
                                    jnp.zeros((L,), jnp.float32))
            pos_v[pl.ds(ci * C, L)] = pos_vec

        issue(0, 0)

        def chunk_body(ci, carry):
            @pl.when(ci % 2 == 0)
            def _():
                @pl.when(ci + 1 < NCHUNK)
                def _():
                    issue(ci + 1, 1)
                drain(0)
                compute(ci, 0)

            @pl.when(ci % 2 == 1)
            def _():
                @pl.when(ci + 1 < NCHUNK)
                def _():
                    issue(ci + 1, 0)
                drain(1)
                compute(ci, 1)
            return carry

        lax.fori_loop(0, NCHUNK, chunk_body, jnp.int32(0))
        pltpu.sync_copy(pos_v, pos_hbm.at[pl.ds(base, BPW)])
        pltpu.sync_copy(neg_v, negout_hbm.at[pl.ds(base * K, PK)])

    return k


def kernel(central_idxs, context_idxs, negative_samples_idxs,
           central_weight, context_weight):
    B, K = negative_samples_idxs.shape
    _, D = central_weight.shape
    cen = central_idxs.astype(jnp.int32)
    ctx = context_idxs.astype(jnp.int32)
    neg = negative_samples_idxs.astype(jnp.int32).reshape(B * K)
    cen_wt = jnp.transpose(central_weight)
    fc = _build_central_kernel(B, D)
    (cemb,) = fc(cen, cen_wt)
    fm = _build_main_kernel(B, K, D)
    pos, negs = fm(ctx, neg, cemb, context_weight)
    return pos.reshape(B, 1), negs.reshape(B, K)
